# Initial kernel scaffold; baseline (speedup 1.0000x reference)
import jax
import jax.numpy as jnp
from jax.experimental import pallas as pl


def kernel(edge_attr, edge_index, nbr_shift, pos, W1, b1, g1, be1, W2, b2, g2, be2, Wout, bout):
    raise NotImplementedError("write your pallas kernel here")


# speedup vs baseline: 4.4623x; 4.4623x over previous
import functools

import jax
import jax.numpy as jnp
from jax import lax
from jax.experimental import pallas as pl
from jax.experimental.pallas import tpu as pltpu
from jax.experimental.pallas import tpu_sc as plsc

_N = 100000
_E = 3200000
_D = 16
_EPS = 1e-5

_RB = 5120
_GRID = _E // _RB

_NC = 2
_NS = 16
_NW = _NC * _NS
_EW = 102400
_EPAD = _NW * _EW
_C = 2048
_S = _C // 128
_NCH = _EW // _C
_NP = 100352
_NROW = _NP // _NS


def _softplus(h):
    return jnp.maximum(h, 0.0) + jnp.log1p(jnp.exp(-jnp.abs(h)))


def _stats1_body(x_ref, w_ref, b_ref, o_ref):
    h = jnp.dot(x_ref[...], w_ref[...], preferred_element_type=jnp.float32)
    h = h + b_ref[...]
    p = jnp.concatenate(
        [jnp.sum(h, axis=0, keepdims=True), jnp.sum(h * h, axis=0, keepdims=True)],
        axis=0)

    @pl.when(pl.program_id(0) == 0)
    def _():
        o_ref[...] = jnp.zeros_like(o_ref)

    o_ref[...] += p


def _stats2_body(x_ref, w1_ref, b1_ref, w2_ref, b2_ref, o_ref):
    h1 = jnp.dot(x_ref[...], w1_ref[...], preferred_element_type=jnp.float32)
    h1 = h1 + b1_ref[...]
    s = _softplus(h1)
    h2 = jnp.dot(s, w2_ref[...], preferred_element_type=jnp.float32) + b2_ref[...]
    p = jnp.concatenate(
        [jnp.sum(h2, axis=0, keepdims=True), jnp.sum(h2 * h2, axis=0, keepdims=True)],
        axis=0)

    @pl.when(pl.program_id(0) == 0)
    def _():
        o_ref[...] = jnp.zeros_like(o_ref)

    o_ref[...] += p


def _scalar_body(x_ref, w1_ref, b1_ref, w2_ref, b2_ref, wo_ref, bo_ref, o_ref):
    h1 = jnp.dot(x_ref[...], w1_ref[...], preferred_element_type=jnp.float32)
    h1 = h1 + b1_ref[...]
    s1 = _softplus(h1)
    h2 = jnp.dot(s1, w2_ref[...], preferred_element_type=jnp.float32) + b2_ref[...]
    s2 = _softplus(h2)
    o_ref[...] = jnp.sum(s2 * wo_ref[...], axis=1) + bo_ref[0, 0]


def _add_body(a_ref, o_ref):
    o_ref[...] = a_ref[0] + a_ref[1]


_whole = lambda shape: pl.BlockSpec(shape, lambda g: tuple(0 for _ in shape))

_stats1 = pl.pallas_call(
    _stats1_body,
    grid=(_GRID,),
    in_specs=[
        pl.BlockSpec((_RB, _D), lambda g: (g, 0)),
        _whole((_D, _D)),
        _whole((1, _D)),
    ],
    out_specs=_whole((2, _D)),
    out_shape=jax.ShapeDtypeStruct((2, _D), jnp.float32),
)

_stats2 = pl.pallas_call(
    _stats2_body,
    grid=(_GRID,),
    in_specs=[
        pl.BlockSpec((_RB, _D), lambda g: (g, 0)),
        _whole((_D, _D)),
        _whole((1, _D)),
        _whole((_D, _D)),
        _whole((1, _D)),
    ],
    out_specs=_whole((2, _D)),
    out_shape=jax.ShapeDtypeStruct((2, _D), jnp.float32),
)

_scalar_pass = pl.pallas_call(
    _scalar_body,
    grid=(_GRID,),
    in_specs=[
        pl.BlockSpec((_RB, _D), lambda g: (g, 0)),
        _whole((_D, _D)),
        _whole((1, _D)),
        _whole((_D, _D)),
        _whole((1, _D)),
        _whole((1, _D)),
        _whole((1, 1)),
    ],
    out_specs=pl.BlockSpec((_RB,), lambda g: (g,)),
    out_shape=jax.ShapeDtypeStruct((_E,), jnp.float32),
)

_ADD_R = _NP * 3 // 128
_add_partials = pl.pallas_call(
    _add_body,
    grid=(1,),
    in_specs=[pl.BlockSpec((2, _ADD_R, 128), lambda g: (0, 0, 0))],
    out_specs=pl.BlockSpec((_ADD_R, 128), lambda g: (0, 0)),
    out_shape=jax.ShapeDtypeStruct((_ADD_R, 128), jnp.float32),
)


def _rsqrt16(s):
    t = s
    r = jnp.full((16,), 1.0, jnp.float32)
    for k in (32, 16, 8, 4, 2, 1):
        big = t >= jnp.float32(4.0 ** k)
        t = jnp.where(big, t * jnp.float32(4.0 ** -k), t)
        r = jnp.where(big, r * jnp.float32(2.0 ** -k), r)
        small = t < jnp.float32(4.0 ** -k)
        t = jnp.where(small, t * jnp.float32(4.0 ** k), t)
        r = jnp.where(small, r * jnp.float32(2.0 ** k), r)
    big = t >= jnp.float32(2.0)
    t = jnp.where(big, t * jnp.float32(0.5), t)
    r = jnp.where(big, r * jnp.float32(0.70710678), r)
    y = jnp.float32(1.53) - jnp.float32(0.4571) * t
    for _ in range(4):
        y = y * (jnp.float32(1.5) - jnp.float32(0.5) * t * y * y)
    return y * r


def _sc_body(i_hbm, j_hbm, shx_hbm, shy_hbm, shz_hbm, scal_hbm,
             px_hbm, py_hbm, pz_hbm, zeros_hbm, out_hbm,
             i_idx, j_idx, xi_b, yi_b, zi_b, xj_b, yj_b, zj_b,
             shx_b, shy_b, shz_b, sc_b, fx_b, fy_b, fz_b,
             ox_sp, oy_sp, oz_sp, sem):
    cid = lax.axis_index("c")
    sid = lax.axis_index("s")
    wid = sid * _NC + cid

    row0 = pl.multiple_of(sid * _NROW, 128)
    pltpu.sync_copy(zeros_hbm.at[pl.ds(row0, _NROW)], ox_sp.at[pl.ds(row0, _NROW)])
    pltpu.sync_copy(zeros_hbm.at[pl.ds(row0, _NROW)], oy_sp.at[pl.ds(row0, _NROW)])
    pltpu.sync_copy(zeros_hbm.at[pl.ds(row0, _NROW)], oz_sp.at[pl.ds(row0, _NROW)])
    plsc.subcore_barrier()

    def _chunk(ch, carry):
        base = pl.multiple_of(wid * _EW + ch * _C, _C)
        brow = pl.multiple_of((wid * _EW + ch * _C) // 128, _S)
        pltpu.sync_copy(i_hbm.at[pl.ds(brow, _S)], i_idx)
        pltpu.sync_copy(j_hbm.at[pl.ds(brow, _S)], j_idx)
        pltpu.sync_copy(shx_hbm.at[pl.ds(base, _C)], shx_b)
        pltpu.sync_copy(shy_hbm.at[pl.ds(base, _C)], shy_b)
        pltpu.sync_copy(shz_hbm.at[pl.ds(base, _C)], shz_b)
        pltpu.sync_copy(scal_hbm.at[pl.ds(base, _C)], sc_b)

        def _fire(s, c2):
            sl = pl.ds(pl.multiple_of(s * 128, 128), 128)
            pltpu.async_copy(px_hbm.at[i_idx.at[s]], xi_b.at[sl], sem)
            pltpu.async_copy(py_hbm.at[i_idx.at[s]], yi_b.at[sl], sem)
            pltpu.async_copy(pz_hbm.at[i_idx.at[s]], zi_b.at[sl], sem)
            pltpu.async_copy(px_hbm.at[j_idx.at[s]], xj_b.at[sl], sem)
            pltpu.async_copy(py_hbm.at[j_idx.at[s]], yj_b.at[sl], sem)
            pltpu.async_copy(pz_hbm.at[j_idx.at[s]], zj_b.at[sl], sem)
            return c2

        lax.fori_loop(0, _S, _fire, 0)

        def _drain(s, c2):
            sl = pl.ds(pl.multiple_of(s * 128, 128), 128)
            pltpu.make_async_copy(px_hbm.at[i_idx.at[s]], xi_b.at[sl], sem).wait()
            pltpu.make_async_copy(py_hbm.at[i_idx.at[s]], yi_b.at[sl], sem).wait()
            pltpu.make_async_copy(pz_hbm.at[i_idx.at[s]], zi_b.at[sl], sem).wait()
            pltpu.make_async_copy(px_hbm.at[j_idx.at[s]], xj_b.at[sl], sem).wait()
            pltpu.make_async_copy(py_hbm.at[j_idx.at[s]], yj_b.at[sl], sem).wait()
            pltpu.make_async_copy(pz_hbm.at[j_idx.at[s]], zj_b.at[sl], sem).wait()
            return c2

        lax.fori_loop(0, _S, _drain, 0)

        def _grp(g, c2):
            sl = pl.ds(pl.multiple_of(g * 16, 16), 16)
            dx = xi_b[sl] + shx_b[sl] - xj_b[sl]
            dy = yi_b[sl] + shy_b[sl] - yj_b[sl]
            dz = zi_b[sl] + shz_b[sl] - zj_b[sl]
            inv = _rsqrt16(dx * dx + dy * dy + dz * dz)
            f = sc_b[sl] * inv
            fx_b[sl] = f * dx
            fy_b[sl] = f * dy
            fz_b[sl] = f * dz
            return c2

        lax.fori_loop(0, _C // 16, _grp, 0)

        def _scat(s, c2):
            sl = pl.ds(pl.multiple_of(s * 128, 128), 128)
            pltpu.sync_copy(fx_b.at[sl], ox_sp.at[i_idx.at[s]], add=True)
            pltpu.sync_copy(fy_b.at[sl], oy_sp.at[i_idx.at[s]], add=True)
            pltpu.sync_copy(fz_b.at[sl], oz_sp.at[i_idx.at[s]], add=True)
            return c2

        lax.fori_loop(0, _S, _scat, 0)
        return carry

    lax.fori_loop(0, _NCH, _chunk, 0)
    plsc.subcore_barrier()
    row1 = pl.multiple_of(sid * _NROW, 128)
    pltpu.sync_copy(ox_sp.at[pl.ds(row1, _NROW)],
                    out_hbm.at[cid, pl.ds(pl.multiple_of(0 * _NP + sid * _NROW, 128), _NROW)])
    pltpu.sync_copy(oy_sp.at[pl.ds(row1, _NROW)],
                    out_hbm.at[cid, pl.ds(pl.multiple_of(1 * _NP + sid * _NROW, 128), _NROW)])
    pltpu.sync_copy(oz_sp.at[pl.ds(row1, _NROW)],
                    out_hbm.at[cid, pl.ds(pl.multiple_of(2 * _NP + sid * _NROW, 128), _NROW)])


@functools.cache
def _sc_scatter_fn():
  c_f32 = pltpu.VMEM((_C,), jnp.float32)
  return pl.kernel(
    _sc_body,
    out_type=jax.ShapeDtypeStruct((_NC, 3 * _NP), jnp.float32),
    mesh=plsc.VectorSubcoreMesh(core_axis_name="c", subcore_axis_name="s",
                                num_cores=_NC, num_subcores=_NS),
    scratch_types=[
        pltpu.VMEM((_S, 128), jnp.int32),
        pltpu.VMEM((_S, 128), jnp.int32),
        c_f32, c_f32, c_f32, c_f32, c_f32, c_f32,
        c_f32, c_f32, c_f32,
        c_f32,
        c_f32, c_f32, c_f32,
        pltpu.VMEM_SHARED((_NP,), jnp.float32),
        pltpu.VMEM_SHARED((_NP,), jnp.float32),
        pltpu.VMEM_SHARED((_NP,), jnp.float32),
        pltpu.SemaphoreType.DMA,
    ],
  )


@jax.jit
def kernel(edge_attr, edge_index, nbr_shift, pos, W1, b1, g1, be1,
           W2, b2, g2, be2, Wout, bout):
    ef = jnp.float32(_E)

    sA = _stats1(edge_attr, W1, b1.reshape(1, _D))
    mean1 = sA[0] / ef
    var1 = sA[1] / ef - mean1 * mean1
    a1 = g1 * lax.rsqrt(var1 + _EPS)
    W1f = W1 * a1[None, :]
    b1f = (b1 * a1 + be1 - mean1 * a1).reshape(1, _D)

    sB = _stats2(edge_attr, W1f, b1f, W2, b2.reshape(1, _D))
    mean2 = sB[0] / ef
    var2 = sB[1] / ef - mean2 * mean2
    a2 = g2 * lax.rsqrt(var2 + _EPS)
    W2f = W2 * a2[None, :]
    b2f = (b2 * a2 + be2 - mean2 * a2).reshape(1, _D)

    scal = _scalar_pass(edge_attr, W1f, b1f, W2f, b2f,
                        Wout.reshape(1, _D), bout.reshape(1, 1))

    pad = _EPAD - _E
    i2 = jnp.pad(edge_index[1], (0, pad)).reshape(-1, 128)
    j2 = jnp.pad(edge_index[0], (0, pad)).reshape(-1, 128)
    shx = jnp.pad(nbr_shift[:, 0], (0, pad), constant_values=1.0)
    shy = jnp.pad(nbr_shift[:, 1], (0, pad))
    shz = jnp.pad(nbr_shift[:, 2], (0, pad))
    scp = jnp.pad(scal, (0, pad))
    zer = jnp.zeros((_NP,), jnp.float32)
    parts = _sc_scatter_fn()(i2, j2, shx, shy, shz, scp,
                             pos[:, 0], pos[:, 1], pos[:, 2], zer)

    out3 = _add_partials(parts.reshape(2, _ADD_R, 128))
    return out3.reshape(3, _NP)[:, :_N].T


# speedup vs baseline: 10.3090x; 2.3102x over previous
import functools

import jax
import jax.numpy as jnp
from jax import lax
from jax.experimental import pallas as pl
from jax.experimental.pallas import tpu as pltpu
from jax.experimental.pallas import tpu_sc as plsc

_N = 100000
_E = 3200000
_D = 16
_EPS = 1e-5

_CB = 25600
_GRID = _E // _CB

_NC = 2
_NS = 16
_NW = _NC * _NS
_EW = 102400
_EPAD = _NW * _EW
_C = 4096
_NCH = _EW // _C
_NP = 100352
_NROW = _NP // _NS


def _softplus(h):
    return jnp.maximum(h, 0.0) + jnp.log1p(jnp.exp(-jnp.abs(h)))


def _stats1_body(x_ref, w_ref, b_ref, o_ref):
    h = jnp.dot(w_ref[...], x_ref[...], preferred_element_type=jnp.float32)
    h = h + b_ref[...]
    p = jnp.stack([jnp.sum(h, axis=1), jnp.sum(h * h, axis=1)], axis=0)

    @pl.when(pl.program_id(0) == 0)
    def _():
        o_ref[...] = jnp.zeros_like(o_ref)

    o_ref[...] += p


def _stats2_body(x_ref, w1_ref, b1_ref, w2_ref, b2_ref, o_ref):
    h1 = jnp.dot(w1_ref[...], x_ref[...], preferred_element_type=jnp.float32)
    h1 = h1 + b1_ref[...]
    s = _softplus(h1)
    h2 = jnp.dot(w2_ref[...], s, preferred_element_type=jnp.float32) + b2_ref[...]
    p = jnp.stack([jnp.sum(h2, axis=1), jnp.sum(h2 * h2, axis=1)], axis=0)

    @pl.when(pl.program_id(0) == 0)
    def _():
        o_ref[...] = jnp.zeros_like(o_ref)

    o_ref[...] += p


def _scalar_body(x_ref, w1_ref, b1_ref, w2_ref, b2_ref, wo_ref, bo_ref, o_ref):
    h1 = jnp.dot(w1_ref[...], x_ref[...], preferred_element_type=jnp.float32)
    h1 = h1 + b1_ref[...]
    s1 = _softplus(h1)
    h2 = jnp.dot(w2_ref[...], s1, preferred_element_type=jnp.float32) + b2_ref[...]
    s2 = _softplus(h2)
    o_ref[...] = jnp.sum(s2 * wo_ref[...], axis=0) + bo_ref[0, 0]


def _add_body(a_ref, o_ref):
    o_ref[...] = a_ref[0] + a_ref[1]


_whole = lambda shape: pl.BlockSpec(shape, lambda g: tuple(0 for _ in shape))

_stats1 = pl.pallas_call(
    _stats1_body,
    grid=(_GRID,),
    in_specs=[
        pl.BlockSpec((_D, _CB), lambda g: (0, g)),
        _whole((_D, _D)),
        _whole((_D, 1)),
    ],
    out_specs=_whole((2, _D)),
    out_shape=jax.ShapeDtypeStruct((2, _D), jnp.float32),
)

_stats2 = pl.pallas_call(
    _stats2_body,
    grid=(_GRID,),
    in_specs=[
        pl.BlockSpec((_D, _CB), lambda g: (0, g)),
        _whole((_D, _D)),
        _whole((_D, 1)),
        _whole((_D, _D)),
        _whole((_D, 1)),
    ],
    out_specs=_whole((2, _D)),
    out_shape=jax.ShapeDtypeStruct((2, _D), jnp.float32),
)

_scalar_pass = pl.pallas_call(
    _scalar_body,
    grid=(_GRID,),
    in_specs=[
        pl.BlockSpec((_D, _CB), lambda g: (0, g)),
        _whole((_D, _D)),
        _whole((_D, 1)),
        _whole((_D, _D)),
        _whole((_D, 1)),
        _whole((_D, 1)),
        _whole((1, 1)),
    ],
    out_specs=pl.BlockSpec((_CB,), lambda g: (g,)),
    out_shape=jax.ShapeDtypeStruct((_E,), jnp.float32),
)

_ADD_R = _NP * 3 // 128
_add_partials = pl.pallas_call(
    _add_body,
    grid=(1,),
    in_specs=[pl.BlockSpec((2, _ADD_R, 128), lambda g: (0, 0, 0))],
    out_specs=pl.BlockSpec((_ADD_R, 128), lambda g: (0, 0)),
    out_shape=jax.ShapeDtypeStruct((_ADD_R, 128), jnp.float32),
)


def _rsqrt16(s):
    t = s
    r = jnp.full((16,), 1.0, jnp.float32)
    for k in (32, 16, 8, 4, 2, 1):
        big = t >= jnp.float32(4.0 ** k)
        t = jnp.where(big, t * jnp.float32(4.0 ** -k), t)
        r = jnp.where(big, r * jnp.float32(2.0 ** -k), r)
        small = t < jnp.float32(4.0 ** -k)
        t = jnp.where(small, t * jnp.float32(4.0 ** k), t)
        r = jnp.where(small, r * jnp.float32(2.0 ** k), r)
    big = t >= jnp.float32(2.0)
    t = jnp.where(big, t * jnp.float32(0.5), t)
    r = jnp.where(big, r * jnp.float32(0.70710678), r)
    y = jnp.float32(1.53) - jnp.float32(0.4571) * t
    for _ in range(4):
        y = y * (jnp.float32(1.5) - jnp.float32(0.5) * t * y * y)
    return y * r


def _sc_body(i_hbm, j_hbm, shx_hbm, shy_hbm, shz_hbm, scal_hbm,
             px_hbm, py_hbm, pz_hbm, zeros_hbm, out_hbm,
             i_idx, j_idx, xi_b, yi_b, zi_b, xj_b, yj_b, zj_b,
             shx_b, shy_b, shz_b, sc_b, fx_b, fy_b, fz_b,
             ox_sp, oy_sp, oz_sp, sem):
    cid = lax.axis_index("c")
    sid = lax.axis_index("s")
    wid = sid * _NC + cid

    row0 = pl.multiple_of(sid * _NROW, 128)
    pltpu.sync_copy(zeros_hbm.at[pl.ds(row0, _NROW)], ox_sp.at[pl.ds(row0, _NROW)])
    pltpu.sync_copy(zeros_hbm.at[pl.ds(row0, _NROW)], oy_sp.at[pl.ds(row0, _NROW)])
    pltpu.sync_copy(zeros_hbm.at[pl.ds(row0, _NROW)], oz_sp.at[pl.ds(row0, _NROW)])
    plsc.subcore_barrier()

    def _chunk(ch, carry):
        base = pl.multiple_of(wid * _EW + ch * _C, _C)
        pltpu.sync_copy(i_hbm.at[pl.ds(base, _C)], i_idx)
        pltpu.sync_copy(j_hbm.at[pl.ds(base, _C)], j_idx)
        pltpu.sync_copy(shx_hbm.at[pl.ds(base, _C)], shx_b)
        pltpu.sync_copy(shy_hbm.at[pl.ds(base, _C)], shy_b)
        pltpu.sync_copy(shz_hbm.at[pl.ds(base, _C)], shz_b)
        pltpu.sync_copy(scal_hbm.at[pl.ds(base, _C)], sc_b)

        pltpu.async_copy(px_hbm.at[i_idx], xi_b, sem)
        pltpu.async_copy(py_hbm.at[i_idx], yi_b, sem)
        pltpu.async_copy(pz_hbm.at[i_idx], zi_b, sem)
        pltpu.async_copy(px_hbm.at[j_idx], xj_b, sem)
        pltpu.async_copy(py_hbm.at[j_idx], yj_b, sem)
        pltpu.async_copy(pz_hbm.at[j_idx], zj_b, sem)
        pltpu.make_async_copy(px_hbm.at[i_idx], xi_b, sem).wait()
        pltpu.make_async_copy(py_hbm.at[i_idx], yi_b, sem).wait()
        pltpu.make_async_copy(pz_hbm.at[i_idx], zi_b, sem).wait()
        pltpu.make_async_copy(px_hbm.at[j_idx], xj_b, sem).wait()
        pltpu.make_async_copy(py_hbm.at[j_idx], yj_b, sem).wait()
        pltpu.make_async_copy(pz_hbm.at[j_idx], zj_b, sem).wait()

        def _grp(g, c2):
            sl = pl.ds(pl.multiple_of(g * 16, 16), 16)
            dx = xi_b[sl] + shx_b[sl] - xj_b[sl]
            dy = yi_b[sl] + shy_b[sl] - yj_b[sl]
            dz = zi_b[sl] + shz_b[sl] - zj_b[sl]
            inv = _rsqrt16(dx * dx + dy * dy + dz * dz)
            f = sc_b[sl] * inv
            fx_b[sl] = f * dx
            fy_b[sl] = f * dy
            fz_b[sl] = f * dz
            return c2

        lax.fori_loop(0, _C // 16, _grp, 0)

        pltpu.sync_copy(fx_b, ox_sp.at[i_idx], add=True)
        pltpu.sync_copy(fy_b, oy_sp.at[i_idx], add=True)
        pltpu.sync_copy(fz_b, oz_sp.at[i_idx], add=True)
        return carry

    lax.fori_loop(0, _NCH, _chunk, 0)
    plsc.subcore_barrier()
    row1 = pl.multiple_of(sid * _NROW, 128)
    pltpu.sync_copy(ox_sp.at[pl.ds(row1, _NROW)],
                    out_hbm.at[cid, pl.ds(pl.multiple_of(0 * _NP + sid * _NROW, 128), _NROW)])
    pltpu.sync_copy(oy_sp.at[pl.ds(row1, _NROW)],
                    out_hbm.at[cid, pl.ds(pl.multiple_of(1 * _NP + sid * _NROW, 128), _NROW)])
    pltpu.sync_copy(oz_sp.at[pl.ds(row1, _NROW)],
                    out_hbm.at[cid, pl.ds(pl.multiple_of(2 * _NP + sid * _NROW, 128), _NROW)])


@functools.cache
def _sc_scatter_fn():
  c_f32 = pltpu.VMEM((_C,), jnp.float32)
  return pl.kernel(
    _sc_body,
    out_type=jax.ShapeDtypeStruct((_NC, 3 * _NP), jnp.float32),
    mesh=plsc.VectorSubcoreMesh(core_axis_name="c", subcore_axis_name="s",
                                num_cores=_NC, num_subcores=_NS),
    scratch_types=[
        pltpu.VMEM((_C,), jnp.int32),
        pltpu.VMEM((_C,), jnp.int32),
        c_f32, c_f32, c_f32, c_f32, c_f32, c_f32,
        c_f32, c_f32, c_f32,
        c_f32,
        c_f32, c_f32, c_f32,
        pltpu.VMEM_SHARED((_NP,), jnp.float32),
        pltpu.VMEM_SHARED((_NP,), jnp.float32),
        pltpu.VMEM_SHARED((_NP,), jnp.float32),
        pltpu.SemaphoreType.DMA,
    ],
  )


@jax.jit
def kernel(edge_attr, edge_index, nbr_shift, pos, W1, b1, g1, be1,
           W2, b2, g2, be2, Wout, bout):
    ef = jnp.float32(_E)
    xT = edge_attr.T

    sA = _stats1(xT, W1.T, b1.reshape(_D, 1))
    mean1 = sA[0] / ef
    var1 = sA[1] / ef - mean1 * mean1
    a1 = g1 * lax.rsqrt(var1 + _EPS)
    W1f = (W1 * a1[None, :]).T
    b1f = (b1 * a1 + be1 - mean1 * a1).reshape(_D, 1)

    sB = _stats2(xT, W1f, b1f, W2.T, b2.reshape(_D, 1))
    mean2 = sB[0] / ef
    var2 = sB[1] / ef - mean2 * mean2
    a2 = g2 * lax.rsqrt(var2 + _EPS)
    W2f = (W2 * a2[None, :]).T
    b2f = (b2 * a2 + be2 - mean2 * a2).reshape(_D, 1)

    scal = _scalar_pass(xT, W1f, b1f, W2f, b2f,
                        Wout.reshape(_D, 1), bout.reshape(1, 1))

    pad = _EPAD - _E
    i1 = jnp.pad(edge_index[1], (0, pad))
    j1 = jnp.pad(edge_index[0], (0, pad))
    shx = jnp.pad(nbr_shift[:, 0], (0, pad), constant_values=1.0)
    shy = jnp.pad(nbr_shift[:, 1], (0, pad))
    shz = jnp.pad(nbr_shift[:, 2], (0, pad))
    scp = jnp.pad(scal, (0, pad))
    zer = jnp.zeros((_NP,), jnp.float32)
    parts = _sc_scatter_fn()(i1, j1, shx, shy, shz, scp,
                             pos[:, 0], pos[:, 1], pos[:, 2], zer)

    out3 = _add_partials(parts.reshape(2, _ADD_R, 128))
    return out3.reshape(3, _NP)[:, :_N].T


# speedup vs baseline: 23.7557x; 2.3044x over previous
import functools

import jax
import jax.numpy as jnp
from jax import lax
from jax.experimental import pallas as pl
from jax.experimental.pallas import tpu as pltpu
from jax.experimental.pallas import tpu_sc as plsc

_N = 100000
_E = 3200000
_D = 16
_EPS = 1e-5

_CB = 25600
_GRID = _E // _CB

_NC = 2
_NS = 16
_NW = _NC * _NS
_EW = 102400
_EPAD = _NW * _EW
_C = 4096
_NCH = _EW // _C
_NP = 100352
_NROW = _NP // _NS


def _softplus(h):
    return jnp.maximum(h, 0.0) + jnp.log1p(jnp.exp(-jnp.abs(h)))


def _stats1_body(x_ref, w_ref, b_ref, o_ref):
    h = jnp.dot(w_ref[...], x_ref[...], preferred_element_type=jnp.float32)
    h = h + b_ref[...]
    p = jnp.stack([jnp.sum(h, axis=1), jnp.sum(h * h, axis=1)], axis=0)

    @pl.when(pl.program_id(0) == 0)
    def _():
        o_ref[...] = jnp.zeros_like(o_ref)

    o_ref[...] += p


def _stats2_body(x_ref, w1_ref, b1_ref, w2_ref, b2_ref, o_ref):
    h1 = jnp.dot(w1_ref[...], x_ref[...], preferred_element_type=jnp.float32)
    h1 = h1 + b1_ref[...]
    s = _softplus(h1)
    h2 = jnp.dot(w2_ref[...], s, preferred_element_type=jnp.float32) + b2_ref[...]
    p = jnp.stack([jnp.sum(h2, axis=1), jnp.sum(h2 * h2, axis=1)], axis=0)

    @pl.when(pl.program_id(0) == 0)
    def _():
        o_ref[...] = jnp.zeros_like(o_ref)

    o_ref[...] += p


def _scalar_body(x_ref, w1_ref, b1_ref, w2_ref, b2_ref, wo_ref, bo_ref, o_ref):
    h1 = jnp.dot(w1_ref[...], x_ref[...], preferred_element_type=jnp.float32)
    h1 = h1 + b1_ref[...]
    s1 = _softplus(h1)
    h2 = jnp.dot(w2_ref[...], s1, preferred_element_type=jnp.float32) + b2_ref[...]
    s2 = _softplus(h2)
    o_ref[...] = jnp.sum(s2 * wo_ref[...], axis=0) + bo_ref[0, 0]


def _add_body(a_ref, o_ref):
    o_ref[...] = a_ref[0] + a_ref[1]


_whole = lambda shape: pl.BlockSpec(shape, lambda g: tuple(0 for _ in shape))

_stats1 = pl.pallas_call(
    _stats1_body,
    grid=(_GRID,),
    in_specs=[
        pl.BlockSpec((_D, _CB), lambda g: (0, g)),
        _whole((_D, _D)),
        _whole((_D, 1)),
    ],
    out_specs=_whole((2, _D)),
    out_shape=jax.ShapeDtypeStruct((2, _D), jnp.float32),
)

_stats2 = pl.pallas_call(
    _stats2_body,
    grid=(_GRID,),
    in_specs=[
        pl.BlockSpec((_D, _CB), lambda g: (0, g)),
        _whole((_D, _D)),
        _whole((_D, 1)),
        _whole((_D, _D)),
        _whole((_D, 1)),
    ],
    out_specs=_whole((2, _D)),
    out_shape=jax.ShapeDtypeStruct((2, _D), jnp.float32),
)

_scalar_pass = pl.pallas_call(
    _scalar_body,
    grid=(_GRID,),
    in_specs=[
        pl.BlockSpec((_D, _CB), lambda g: (0, g)),
        _whole((_D, _D)),
        _whole((_D, 1)),
        _whole((_D, _D)),
        _whole((_D, 1)),
        _whole((_D, 1)),
        _whole((1, 1)),
    ],
    out_specs=pl.BlockSpec((_CB,), lambda g: (g,)),
    out_shape=jax.ShapeDtypeStruct((_E,), jnp.float32),
)

_ADD_R = _NP * 3 // 128
_add_partials = pl.pallas_call(
    _add_body,
    grid=(1,),
    in_specs=[pl.BlockSpec((2, _ADD_R, 128), lambda g: (0, 0, 0))],
    out_specs=pl.BlockSpec((_ADD_R, 128), lambda g: (0, 0)),
    out_shape=jax.ShapeDtypeStruct((_ADD_R, 128), jnp.float32),
)


def _rsqrt16(s):
    t = s
    r = jnp.full((16,), 1.0, jnp.float32)
    for k in (32, 16, 8, 4, 2, 1):
        big = t >= jnp.float32(4.0 ** k)
        t = jnp.where(big, t * jnp.float32(4.0 ** -k), t)
        r = jnp.where(big, r * jnp.float32(2.0 ** -k), r)
        small = t < jnp.float32(4.0 ** -k)
        t = jnp.where(small, t * jnp.float32(4.0 ** k), t)
        r = jnp.where(small, r * jnp.float32(2.0 ** k), r)
    big = t >= jnp.float32(2.0)
    t = jnp.where(big, t * jnp.float32(0.5), t)
    r = jnp.where(big, r * jnp.float32(0.70710678), r)
    y = jnp.float32(1.53) - jnp.float32(0.4571) * t
    for _ in range(4):
        y = y * (jnp.float32(1.5) - jnp.float32(0.5) * t * y * y)
    return y * r


def _sc_body(i_hbm, j_hbm, shx_hbm, shy_hbm, shz_hbm, scal_hbm,
             px_hbm, py_hbm, pz_hbm, zeros_hbm, out_hbm,
             i_idx, j_idx, px_sp, py_sp, pz_sp, xi_b, yi_b, zi_b, xj_b, yj_b, zj_b,
             shx_b, shy_b, shz_b, sc_b, fx_b, fy_b, fz_b,
             ox_sp, oy_sp, oz_sp, sem):
    cid = lax.axis_index("c")
    sid = lax.axis_index("s")
    wid = sid * _NC + cid

    row0 = pl.multiple_of(sid * _NROW, 128)
    pltpu.sync_copy(zeros_hbm.at[pl.ds(row0, _NROW)], ox_sp.at[pl.ds(row0, _NROW)])
    pltpu.sync_copy(zeros_hbm.at[pl.ds(row0, _NROW)], oy_sp.at[pl.ds(row0, _NROW)])
    pltpu.sync_copy(zeros_hbm.at[pl.ds(row0, _NROW)], oz_sp.at[pl.ds(row0, _NROW)])
    pltpu.sync_copy(px_hbm.at[pl.ds(row0, _NROW)], px_sp.at[pl.ds(row0, _NROW)])
    pltpu.sync_copy(py_hbm.at[pl.ds(row0, _NROW)], py_sp.at[pl.ds(row0, _NROW)])
    pltpu.sync_copy(pz_hbm.at[pl.ds(row0, _NROW)], pz_sp.at[pl.ds(row0, _NROW)])
    plsc.subcore_barrier()

    def _chunk(ch, carry):
        base = pl.multiple_of(wid * _EW + ch * _C, _C)
        pltpu.sync_copy(i_hbm.at[pl.ds(base, _C)], i_idx)
        pltpu.sync_copy(j_hbm.at[pl.ds(base, _C)], j_idx)
        pltpu.sync_copy(shx_hbm.at[pl.ds(base, _C)], shx_b)
        pltpu.sync_copy(shy_hbm.at[pl.ds(base, _C)], shy_b)
        pltpu.sync_copy(shz_hbm.at[pl.ds(base, _C)], shz_b)
        pltpu.sync_copy(scal_hbm.at[pl.ds(base, _C)], sc_b)

        pltpu.async_copy(px_sp.at[i_idx], xi_b, sem)
        pltpu.async_copy(py_sp.at[i_idx], yi_b, sem)
        pltpu.async_copy(pz_sp.at[i_idx], zi_b, sem)
        pltpu.async_copy(px_sp.at[j_idx], xj_b, sem)
        pltpu.async_copy(py_sp.at[j_idx], yj_b, sem)
        pltpu.async_copy(pz_sp.at[j_idx], zj_b, sem)
        pltpu.make_async_copy(px_sp.at[i_idx], xi_b, sem).wait()
        pltpu.make_async_copy(py_sp.at[i_idx], yi_b, sem).wait()
        pltpu.make_async_copy(pz_sp.at[i_idx], zi_b, sem).wait()
        pltpu.make_async_copy(px_sp.at[j_idx], xj_b, sem).wait()
        pltpu.make_async_copy(py_sp.at[j_idx], yj_b, sem).wait()
        pltpu.make_async_copy(pz_sp.at[j_idx], zj_b, sem).wait()

        def _grp(g, c2):
            sl = pl.ds(pl.multiple_of(g * 16, 16), 16)
            dx = xi_b[sl] + shx_b[sl] - xj_b[sl]
            dy = yi_b[sl] + shy_b[sl] - yj_b[sl]
            dz = zi_b[sl] + shz_b[sl] - zj_b[sl]
            inv = _rsqrt16(dx * dx + dy * dy + dz * dz)
            f = sc_b[sl] * inv
            fx_b[sl] = f * dx
            fy_b[sl] = f * dy
            fz_b[sl] = f * dz
            return c2

        lax.fori_loop(0, _C // 16, _grp, 0)

        pltpu.sync_copy(fx_b, ox_sp.at[i_idx], add=True)
        pltpu.sync_copy(fy_b, oy_sp.at[i_idx], add=True)
        pltpu.sync_copy(fz_b, oz_sp.at[i_idx], add=True)
        return carry

    lax.fori_loop(0, _NCH, _chunk, 0)
    plsc.subcore_barrier()
    row1 = pl.multiple_of(sid * _NROW, 128)
    pltpu.sync_copy(ox_sp.at[pl.ds(row1, _NROW)],
                    out_hbm.at[cid, pl.ds(pl.multiple_of(0 * _NP + sid * _NROW, 128), _NROW)])
    pltpu.sync_copy(oy_sp.at[pl.ds(row1, _NROW)],
                    out_hbm.at[cid, pl.ds(pl.multiple_of(1 * _NP + sid * _NROW, 128), _NROW)])
    pltpu.sync_copy(oz_sp.at[pl.ds(row1, _NROW)],
                    out_hbm.at[cid, pl.ds(pl.multiple_of(2 * _NP + sid * _NROW, 128), _NROW)])


@functools.cache
def _sc_scatter_fn():
  c_f32 = pltpu.VMEM((_C,), jnp.float32)
  return pl.kernel(
    _sc_body,
    out_type=jax.ShapeDtypeStruct((_NC, 3 * _NP), jnp.float32),
    mesh=plsc.VectorSubcoreMesh(core_axis_name="c", subcore_axis_name="s",
                                num_cores=_NC, num_subcores=_NS),
    scratch_types=[
        pltpu.VMEM((_C,), jnp.int32),
        pltpu.VMEM((_C,), jnp.int32),
        pltpu.VMEM_SHARED((_NP,), jnp.float32),
        pltpu.VMEM_SHARED((_NP,), jnp.float32),
        pltpu.VMEM_SHARED((_NP,), jnp.float32),
        c_f32, c_f32, c_f32, c_f32, c_f32, c_f32,
        c_f32, c_f32, c_f32,
        c_f32,
        c_f32, c_f32, c_f32,
        pltpu.VMEM_SHARED((_NP,), jnp.float32),
        pltpu.VMEM_SHARED((_NP,), jnp.float32),
        pltpu.VMEM_SHARED((_NP,), jnp.float32),
        pltpu.SemaphoreType.DMA,
    ],
  )


@jax.jit
def kernel(edge_attr, edge_index, nbr_shift, pos, W1, b1, g1, be1,
           W2, b2, g2, be2, Wout, bout):
    ef = jnp.float32(_E)
    xT = edge_attr.T

    sA = _stats1(xT, W1.T, b1.reshape(_D, 1))
    mean1 = sA[0] / ef
    var1 = sA[1] / ef - mean1 * mean1
    a1 = g1 * lax.rsqrt(var1 + _EPS)
    W1f = (W1 * a1[None, :]).T
    b1f = (b1 * a1 + be1 - mean1 * a1).reshape(_D, 1)

    sB = _stats2(xT, W1f, b1f, W2.T, b2.reshape(_D, 1))
    mean2 = sB[0] / ef
    var2 = sB[1] / ef - mean2 * mean2
    a2 = g2 * lax.rsqrt(var2 + _EPS)
    W2f = (W2 * a2[None, :]).T
    b2f = (b2 * a2 + be2 - mean2 * a2).reshape(_D, 1)

    scal = _scalar_pass(xT, W1f, b1f, W2f, b2f,
                        Wout.reshape(_D, 1), bout.reshape(1, 1))

    pad = _EPAD - _E
    i1 = jnp.pad(edge_index[1], (0, pad))
    j1 = jnp.pad(edge_index[0], (0, pad))
    shx = jnp.pad(nbr_shift[:, 0], (0, pad), constant_values=1.0)
    shy = jnp.pad(nbr_shift[:, 1], (0, pad))
    shz = jnp.pad(nbr_shift[:, 2], (0, pad))
    scp = jnp.pad(scal, (0, pad))
    zer = jnp.zeros((_NP,), jnp.float32)
    posp = jnp.pad(pos, ((0, _NP - _N), (0, 0)))
    parts = _sc_scatter_fn()(i1, j1, shx, shy, shz, scp,
                             posp[:, 0], posp[:, 1], posp[:, 2], zer)

    out3 = _add_partials(parts.reshape(2, _ADD_R, 128))
    return out3.reshape(3, _NP)[:, :_N].T


# speedup vs baseline: 29.2295x; 1.2304x over previous
import functools

import jax
import jax.numpy as jnp
from jax import lax
from jax.experimental import pallas as pl
from jax.experimental.pallas import tpu as pltpu
from jax.experimental.pallas import tpu_sc as plsc

_N = 100000
_E = 3200000
_D = 16
_EPS = 1e-5

_CB = 25600
_GRID = _E // _CB

_NC = 2
_NS = 16
_NW = _NC * _NS
_EW = 102400
_EPAD = _NW * _EW
_C = 2048
_NCH = _EW // _C
_NP = 100352
_NROW = _NP // _NS


def _softplus(h):
    return jnp.maximum(h, 0.0) + jnp.log1p(jnp.exp(-jnp.abs(h)))


def _stats1_body(x_ref, w_ref, b_ref, o_ref):
    h = jnp.dot(w_ref[...], x_ref[...], preferred_element_type=jnp.float32)
    h = h + b_ref[...]
    p = jnp.stack([jnp.sum(h, axis=1), jnp.sum(h * h, axis=1)], axis=0)

    @pl.when(pl.program_id(0) == 0)
    def _():
        o_ref[...] = jnp.zeros_like(o_ref)

    o_ref[...] += p


def _stats2_body(x_ref, w1_ref, b1_ref, w2_ref, b2_ref, o_ref):
    h1 = jnp.dot(w1_ref[...], x_ref[...], preferred_element_type=jnp.float32)
    h1 = h1 + b1_ref[...]
    s = _softplus(h1)
    h2 = jnp.dot(w2_ref[...], s, preferred_element_type=jnp.float32) + b2_ref[...]
    p = jnp.stack([jnp.sum(h2, axis=1), jnp.sum(h2 * h2, axis=1)], axis=0)

    @pl.when(pl.program_id(0) == 0)
    def _():
        o_ref[...] = jnp.zeros_like(o_ref)

    o_ref[...] += p


def _scalar_body(x_ref, w1_ref, b1_ref, w2_ref, b2_ref, wo_ref, bo_ref, o_ref):
    h1 = jnp.dot(w1_ref[...], x_ref[...], preferred_element_type=jnp.float32)
    h1 = h1 + b1_ref[...]
    s1 = _softplus(h1)
    h2 = jnp.dot(w2_ref[...], s1, preferred_element_type=jnp.float32) + b2_ref[...]
    s2 = _softplus(h2)
    o_ref[...] = jnp.sum(s2 * wo_ref[...], axis=0) + bo_ref[0, 0]


def _add_body(a_ref, o_ref):
    o_ref[...] = a_ref[0] + a_ref[1]


_whole = lambda shape: pl.BlockSpec(shape, lambda g: tuple(0 for _ in shape))

_stats1 = pl.pallas_call(
    _stats1_body,
    grid=(_GRID,),
    in_specs=[
        pl.BlockSpec((_D, _CB), lambda g: (0, g)),
        _whole((_D, _D)),
        _whole((_D, 1)),
    ],
    out_specs=_whole((2, _D)),
    out_shape=jax.ShapeDtypeStruct((2, _D), jnp.float32),
)

_stats2 = pl.pallas_call(
    _stats2_body,
    grid=(_GRID,),
    in_specs=[
        pl.BlockSpec((_D, _CB), lambda g: (0, g)),
        _whole((_D, _D)),
        _whole((_D, 1)),
        _whole((_D, _D)),
        _whole((_D, 1)),
    ],
    out_specs=_whole((2, _D)),
    out_shape=jax.ShapeDtypeStruct((2, _D), jnp.float32),
)

_scalar_pass = pl.pallas_call(
    _scalar_body,
    grid=(_GRID,),
    in_specs=[
        pl.BlockSpec((_D, _CB), lambda g: (0, g)),
        _whole((_D, _D)),
        _whole((_D, 1)),
        _whole((_D, _D)),
        _whole((_D, 1)),
        _whole((_D, 1)),
        _whole((1, 1)),
    ],
    out_specs=pl.BlockSpec((_CB,), lambda g: (g,)),
    out_shape=jax.ShapeDtypeStruct((_E,), jnp.float32),
)

_ADD_R = _NP * 3 // 128
_add_partials = pl.pallas_call(
    _add_body,
    grid=(1,),
    in_specs=[pl.BlockSpec((2, _ADD_R, 128), lambda g: (0, 0, 0))],
    out_specs=pl.BlockSpec((_ADD_R, 128), lambda g: (0, 0)),
    out_shape=jax.ShapeDtypeStruct((_ADD_R, 128), jnp.float32),
)


def _rsqrt16(s):
    t = s
    r = jnp.full((16,), 1.0, jnp.float32)
    for k in (32, 16, 8, 4, 2, 1):
        big = t >= jnp.float32(4.0 ** k)
        t = jnp.where(big, t * jnp.float32(4.0 ** -k), t)
        r = jnp.where(big, r * jnp.float32(2.0 ** -k), r)
        small = t < jnp.float32(4.0 ** -k)
        t = jnp.where(small, t * jnp.float32(4.0 ** k), t)
        r = jnp.where(small, r * jnp.float32(2.0 ** k), r)
    big = t >= jnp.float32(2.0)
    t = jnp.where(big, t * jnp.float32(0.5), t)
    r = jnp.where(big, r * jnp.float32(0.70710678), r)
    y = jnp.float32(1.53) - jnp.float32(0.4571) * t
    for _ in range(4):
        y = y * (jnp.float32(1.5) - jnp.float32(0.5) * t * y * y)
    return y * r


def _sc_gather_grp(ih, jh, sxh, syh, szh, sch, base, B, sem):
    pltpu.async_copy(ih.at[pl.ds(base, _C)], B[0], sem)
    pltpu.async_copy(jh.at[pl.ds(base, _C)], B[1], sem)
    pltpu.async_copy(sxh.at[pl.ds(base, _C)], B[2], sem)
    pltpu.async_copy(syh.at[pl.ds(base, _C)], B[3], sem)
    pltpu.async_copy(szh.at[pl.ds(base, _C)], B[4], sem)
    pltpu.async_copy(sch.at[pl.ds(base, _C)], B[5], sem)


def _sc_drain_grp(ih, jh, sxh, syh, szh, sch, base, B, sem):
    pltpu.make_async_copy(ih.at[pl.ds(base, _C)], B[0], sem).wait()
    pltpu.make_async_copy(jh.at[pl.ds(base, _C)], B[1], sem).wait()
    pltpu.make_async_copy(sxh.at[pl.ds(base, _C)], B[2], sem).wait()
    pltpu.make_async_copy(syh.at[pl.ds(base, _C)], B[3], sem).wait()
    pltpu.make_async_copy(szh.at[pl.ds(base, _C)], B[4], sem).wait()
    pltpu.make_async_copy(sch.at[pl.ds(base, _C)], B[5], sem).wait()


def _sc_fire_gathers(px_sp, py_sp, pz_sp, B, sem):
    pltpu.async_copy(px_sp.at[B[0]], B[6], sem)
    pltpu.async_copy(py_sp.at[B[0]], B[7], sem)
    pltpu.async_copy(pz_sp.at[B[0]], B[8], sem)
    pltpu.async_copy(px_sp.at[B[1]], B[9], sem)
    pltpu.async_copy(py_sp.at[B[1]], B[10], sem)
    pltpu.async_copy(pz_sp.at[B[1]], B[11], sem)


def _sc_drain_gathers(px_sp, py_sp, pz_sp, B, sem):
    pltpu.make_async_copy(px_sp.at[B[0]], B[6], sem).wait()
    pltpu.make_async_copy(py_sp.at[B[0]], B[7], sem).wait()
    pltpu.make_async_copy(pz_sp.at[B[0]], B[8], sem).wait()
    pltpu.make_async_copy(px_sp.at[B[1]], B[9], sem).wait()
    pltpu.make_async_copy(py_sp.at[B[1]], B[10], sem).wait()
    pltpu.make_async_copy(pz_sp.at[B[1]], B[11], sem).wait()


def _sc_compute(B):
    def _grp(g, c2):
        sl = pl.ds(pl.multiple_of(g * 16, 16), 16)
        dx = B[6][sl] + B[2][sl] - B[9][sl]
        dy = B[7][sl] + B[3][sl] - B[10][sl]
        dz = B[8][sl] + B[4][sl] - B[11][sl]
        inv = _rsqrt16(dx * dx + dy * dy + dz * dz)
        f = B[5][sl] * inv
        B[12][sl] = f * dx
        B[13][sl] = f * dy
        B[14][sl] = f * dz
        return c2

    lax.fori_loop(0, _C // 16, _grp, 0)


def _sc_body(i_hbm, j_hbm, shx_hbm, shy_hbm, shz_hbm, scal_hbm,
             px_hbm, py_hbm, pz_hbm, zeros_hbm, out_hbm,
             ii0, jj0, sx0, sy0, sz0, sc0, xi0, yi0, zi0, xj0, yj0, zj0,
             fx0, fy0, fz0,
             ii1, jj1, sx1, sy1, sz1, sc1, xi1, yi1, zi1, xj1, yj1, zj1,
             fx1, fy1, fz1,
             px_sp, py_sp, pz_sp, ox_sp, oy_sp, oz_sp,
             lsem0, lsem1, gsem0, gsem1):
    cid = lax.axis_index("c")
    sid = lax.axis_index("s")
    wid = sid * _NC + cid
    BUFS = ((ii0, jj0, sx0, sy0, sz0, sc0, xi0, yi0, zi0, xj0, yj0, zj0,
             fx0, fy0, fz0),
            (ii1, jj1, sx1, sy1, sz1, sc1, xi1, yi1, zi1, xj1, yj1, zj1,
             fx1, fy1, fz1))
    LSEM = (lsem0, lsem1)
    GSEM = (gsem0, gsem1)
    lin = (i_hbm, j_hbm, shx_hbm, shy_hbm, shz_hbm, scal_hbm)

    row0 = pl.multiple_of(sid * _NROW, 128)
    pltpu.sync_copy(zeros_hbm.at[pl.ds(row0, _NROW)], ox_sp.at[pl.ds(row0, _NROW)])
    pltpu.sync_copy(zeros_hbm.at[pl.ds(row0, _NROW)], oy_sp.at[pl.ds(row0, _NROW)])
    pltpu.sync_copy(zeros_hbm.at[pl.ds(row0, _NROW)], oz_sp.at[pl.ds(row0, _NROW)])
    pltpu.sync_copy(px_hbm.at[pl.ds(row0, _NROW)], px_sp.at[pl.ds(row0, _NROW)])
    pltpu.sync_copy(py_hbm.at[pl.ds(row0, _NROW)], py_sp.at[pl.ds(row0, _NROW)])
    pltpu.sync_copy(pz_hbm.at[pl.ds(row0, _NROW)], pz_sp.at[pl.ds(row0, _NROW)])
    plsc.subcore_barrier()

    def _base(ch):
        return pl.multiple_of(wid * _EW + ch * _C, _C)

    _sc_gather_grp(*lin, _base(0), BUFS[0], LSEM[0])
    _sc_drain_grp(*lin, _base(0), BUFS[0], LSEM[0])
    _sc_fire_gathers(px_sp, py_sp, pz_sp, BUFS[0], GSEM[0])
    _sc_gather_grp(*lin, _base(1), BUFS[1], LSEM[1])

    def _pair(it, carry):
        last = it >= _NCH // 2 - 1
        for b in (0, 1):
            ch = it * 2 + b
            nb = 1 - b
            B = BUFS[b]
            NB = BUFS[nb]

            @pl.when(jnp.logical_or(b == 0, jnp.logical_not(last)))
            def _():
                _sc_drain_grp(*lin, _base(ch + 1), NB, LSEM[nb])
                _sc_fire_gathers(px_sp, py_sp, pz_sp, NB, GSEM[nb])

            _sc_drain_gathers(px_sp, py_sp, pz_sp, B, GSEM[b])
            _sc_compute(B)
            pltpu.sync_copy(B[12], ox_sp.at[B[0]], add=True)
            pltpu.sync_copy(B[13], oy_sp.at[B[0]], add=True)
            pltpu.sync_copy(B[14], oz_sp.at[B[0]], add=True)

            @pl.when(jnp.logical_not(last))
            def _():
                _sc_gather_grp(*lin, _base(ch + 2), B, LSEM[b])

        return carry

    lax.fori_loop(0, _NCH // 2, _pair, 0)
    plsc.subcore_barrier()
    row1 = pl.multiple_of(sid * _NROW, 128)
    pltpu.sync_copy(ox_sp.at[pl.ds(row1, _NROW)],
                    out_hbm.at[cid, pl.ds(pl.multiple_of(0 * _NP + sid * _NROW, 128), _NROW)])
    pltpu.sync_copy(oy_sp.at[pl.ds(row1, _NROW)],
                    out_hbm.at[cid, pl.ds(pl.multiple_of(1 * _NP + sid * _NROW, 128), _NROW)])
    pltpu.sync_copy(oz_sp.at[pl.ds(row1, _NROW)],
                    out_hbm.at[cid, pl.ds(pl.multiple_of(2 * _NP + sid * _NROW, 128), _NROW)])


@functools.cache
def _sc_scatter_fn():
  c_f32 = pltpu.VMEM((_C,), jnp.float32)
  c_i32 = pltpu.VMEM((_C,), jnp.int32)
  one_set = [c_i32, c_i32] + [c_f32] * 13
  return pl.kernel(
    _sc_body,
    out_type=jax.ShapeDtypeStruct((_NC, 3 * _NP), jnp.float32),
    mesh=plsc.VectorSubcoreMesh(core_axis_name="c", subcore_axis_name="s",
                                num_cores=_NC, num_subcores=_NS),
    scratch_types=one_set + one_set + [
        pltpu.VMEM_SHARED((_NP,), jnp.float32),
        pltpu.VMEM_SHARED((_NP,), jnp.float32),
        pltpu.VMEM_SHARED((_NP,), jnp.float32),
        pltpu.VMEM_SHARED((_NP,), jnp.float32),
        pltpu.VMEM_SHARED((_NP,), jnp.float32),
        pltpu.VMEM_SHARED((_NP,), jnp.float32),
        pltpu.SemaphoreType.DMA,
        pltpu.SemaphoreType.DMA,
        pltpu.SemaphoreType.DMA,
        pltpu.SemaphoreType.DMA,
    ],
  )


@jax.jit
def kernel(edge_attr, edge_index, nbr_shift, pos, W1, b1, g1, be1,
           W2, b2, g2, be2, Wout, bout):
    ef = jnp.float32(_E)
    xT = edge_attr.T

    sA = _stats1(xT, W1.T, b1.reshape(_D, 1))
    mean1 = sA[0] / ef
    var1 = sA[1] / ef - mean1 * mean1
    a1 = g1 * lax.rsqrt(var1 + _EPS)
    W1f = (W1 * a1[None, :]).T
    b1f = (b1 * a1 + be1 - mean1 * a1).reshape(_D, 1)

    sB = _stats2(xT, W1f, b1f, W2.T, b2.reshape(_D, 1))
    mean2 = sB[0] / ef
    var2 = sB[1] / ef - mean2 * mean2
    a2 = g2 * lax.rsqrt(var2 + _EPS)
    W2f = (W2 * a2[None, :]).T
    b2f = (b2 * a2 + be2 - mean2 * a2).reshape(_D, 1)

    scal = _scalar_pass(xT, W1f, b1f, W2f, b2f,
                        Wout.reshape(_D, 1), bout.reshape(1, 1))

    pad = _EPAD - _E
    i1 = jnp.pad(edge_index[1], (0, pad))
    j1 = jnp.pad(edge_index[0], (0, pad))
    shx = jnp.pad(nbr_shift[:, 0], (0, pad), constant_values=1.0)
    shy = jnp.pad(nbr_shift[:, 1], (0, pad))
    shz = jnp.pad(nbr_shift[:, 2], (0, pad))
    scp = jnp.pad(scal, (0, pad))
    zer = jnp.zeros((_NP,), jnp.float32)
    posp = jnp.pad(pos, ((0, _NP - _N), (0, 0)))
    parts = _sc_scatter_fn()(i1, j1, shx, shy, shz, scp,
                             posp[:, 0], posp[:, 1], posp[:, 2], zer)

    out3 = _add_partials(parts.reshape(2, _ADD_R, 128))
    return out3.reshape(3, _NP)[:, :_N].T
